# baseline (speedup 1.0000x reference)
import functools

import jax
import jax.numpy as jnp
from jax import lax
from jax.experimental import pallas as pl
from jax.experimental.pallas import tpu as pltpu
from jax.experimental.pallas import tpu_sc as plsc

_L = 16


def _stitch_body(n_rows, n_cols, m0, part_hbm, idx0_hbm, data_hbm, out_hbm,
                 part_v, idx0_v, data_v, nz_v, out_v):
    pltpu.sync_copy(part_hbm, part_v)
    pltpu.sync_copy(idx0_hbm, idx0_v)
    pltpu.sync_copy(data_hbm, data_v)

    lanes = lax.iota(jnp.int32, _L)
    zeros = jnp.zeros((_L,), jnp.float32)

    row = jnp.minimum(lanes // n_cols, m0 - 1)
    part = plsc.load_gather(part_v, [jnp.minimum(lanes, n_rows - 1)])
    in_part0 = (part == 0) & (lanes < n_rows)
    rank = plsc.cumsum(jnp.where(in_part0, 1, 0)) - 1
    plsc.store_scatter(nz_v, [jnp.minimum(lanes, m0 - 1)],
                       jnp.zeros((_L,), jnp.int32), mask=lanes < m0)
    plsc.store_scatter(nz_v, [rank], lanes, mask=in_part0)

    col = lanes - (lanes // n_cols) * n_cols
    valid = lanes < m0 * n_cols

    src_row = plsc.load_gather(nz_v, [row])
    part0 = plsc.load_gather(data_v, [src_row, col], mask=valid)

    dst_row = plsc.load_gather(idx0_v, [row])
    dst_ok = valid & (dst_row >= 0) & (dst_row < n_rows)
    dst_row = jnp.clip(dst_row, 0, n_rows - 1)
    plsc.store_scatter(out_v, [jnp.minimum(lanes, n_rows * n_cols - 1)],
                       zeros, mask=lanes < n_rows * n_cols)
    plsc.store_scatter(out_v, [dst_row * n_cols + col], part0, mask=dst_ok)

    pltpu.sync_copy(out_v, out_hbm)


def kernel(data, partitions, index0, index1):
    n_rows, n_cols = data.shape
    m0 = index0.shape[0]
    assert n_rows * n_cols <= _L and m0 * n_cols <= _L
    assert index1.shape[0] == 0

    body = functools.partial(_stitch_body, n_rows, n_cols, m0)
    out = pl.kernel(
        body,
        out_type=jax.ShapeDtypeStruct((n_rows * n_cols,), jnp.float32),
        mesh=plsc.VectorSubcoreMesh(
            core_axis_name="c", subcore_axis_name="s",
            num_cores=1, num_subcores=1,
        ),
        scratch_types=[
            pltpu.VMEM((n_rows,), jnp.int32),
            pltpu.VMEM((m0,), jnp.int32),
            pltpu.VMEM((n_rows, n_cols), jnp.float32),
            pltpu.VMEM((m0,), jnp.int32),
            pltpu.VMEM((n_rows * n_cols,), jnp.float32),
        ],
        compiler_params=pltpu.CompilerParams(
            needs_layout_passes=False,
            disable_bounds_checks=True,
            disable_semaphore_checks=True,
            skip_device_barrier=True,
        ),
    )(partitions, index0, data)
    return out.reshape(n_rows, n_cols)


# speedup vs baseline: 1.0543x; 1.0543x over previous
import functools

import jax
import jax.numpy as jnp
from jax import lax
from jax.experimental import pallas as pl
from jax.experimental.pallas import tpu as pltpu
from jax.experimental.pallas import tpu_sc as plsc

_L = 16


def _stitch_body(n_rows, n_cols, m0, part_hbm, idx0_hbm, data_hbm, out_hbm,
                 part_v, idx0_v, data_v, nz_v, out_v, sem):
    copies = [pltpu.async_copy(part_hbm, part_v, sem),
              pltpu.async_copy(idx0_hbm, idx0_v, sem),
              pltpu.async_copy(data_hbm, data_v, sem)]
    for c in copies:
        c.wait()

    lanes = lax.iota(jnp.int32, _L)
    zeros = jnp.zeros((_L,), jnp.float32)

    row = jnp.minimum(lanes // n_cols, m0 - 1)
    part = plsc.load_gather(part_v, [jnp.minimum(lanes, n_rows - 1)])
    in_part0 = (part == 0) & (lanes < n_rows)
    rank = plsc.cumsum(jnp.where(in_part0, 1, 0)) - 1
    plsc.store_scatter(nz_v, [jnp.minimum(lanes, m0 - 1)],
                       jnp.zeros((_L,), jnp.int32), mask=lanes < m0)
    plsc.store_scatter(nz_v, [rank], lanes, mask=in_part0)

    col = lanes - (lanes // n_cols) * n_cols
    valid = lanes < m0 * n_cols

    src_row = plsc.load_gather(nz_v, [row])
    part0 = plsc.load_gather(data_v, [src_row, col], mask=valid)

    dst_row = plsc.load_gather(idx0_v, [row])
    dst_ok = valid & (dst_row >= 0) & (dst_row < n_rows)
    dst_row = jnp.clip(dst_row, 0, n_rows - 1)
    plsc.store_scatter(out_v, [jnp.minimum(lanes, n_rows * n_cols - 1)],
                       zeros, mask=lanes < n_rows * n_cols)
    plsc.store_scatter(out_v, [dst_row * n_cols + col], part0, mask=dst_ok)

    pltpu.sync_copy(out_v, out_hbm)


def kernel(data, partitions, index0, index1):
    n_rows, n_cols = data.shape
    m0 = index0.shape[0]
    assert n_rows * n_cols <= _L and m0 * n_cols <= _L
    assert index1.shape[0] == 0

    body = functools.partial(_stitch_body, n_rows, n_cols, m0)
    out = pl.kernel(
        body,
        out_type=jax.ShapeDtypeStruct((n_rows * n_cols,), jnp.float32),
        mesh=plsc.VectorSubcoreMesh(
            core_axis_name="c", subcore_axis_name="s",
            num_cores=1, num_subcores=1,
        ),
        scratch_types=[
            pltpu.VMEM((n_rows,), jnp.int32),
            pltpu.VMEM((m0,), jnp.int32),
            pltpu.VMEM((n_rows, n_cols), jnp.float32),
            pltpu.VMEM((m0,), jnp.int32),
            pltpu.VMEM((n_rows * n_cols,), jnp.float32),
            pltpu.SemaphoreType.DMA,
        ],
        compiler_params=pltpu.CompilerParams(
            needs_layout_passes=False,
            disable_bounds_checks=True,
            disable_semaphore_checks=True,
            skip_device_barrier=True,
        ),
    )(partitions, index0, data)
    return out.reshape(n_rows, n_cols)


# speedup vs baseline: 1.0563x; 1.0019x over previous
import functools

import jax
import jax.numpy as jnp
from jax import lax
from jax.experimental import pallas as pl
from jax.experimental.pallas import tpu as pltpu
from jax.experimental.pallas import tpu_sc as plsc

_L = 16


def _stitch_body(n_rows, n_cols, m0, part_hbm, idx0_hbm, data_hbm, out_hbm,
                 part_v, idx0_v, data_v, nz_v, out_v, sem):
    copies = [pltpu.async_copy(part_hbm, part_v, sem),
              pltpu.async_copy(idx0_hbm, idx0_v, sem),
              pltpu.async_copy(data_hbm, data_v, sem)]
    for c in copies:
        c.wait()

    lanes = lax.iota(jnp.int32, _L)
    zeros = jnp.zeros((_L,), jnp.float32)

    row = jnp.minimum(lanes // n_cols, m0 - 1)
    part = plsc.load_gather(part_v, [jnp.minimum(lanes, n_rows - 1)])
    in_part0 = (part == 0) & (lanes < n_rows)
    rank = plsc.cumsum(jnp.where(in_part0, 1, 0)) - 1
    plsc.store_scatter(nz_v, [jnp.minimum(lanes, m0 - 1)],
                       jnp.zeros((_L,), jnp.int32), mask=lanes < m0)
    plsc.store_scatter(nz_v, [rank], lanes, mask=in_part0 & (rank < m0))

    col = lanes - (lanes // n_cols) * n_cols
    valid = lanes < m0 * n_cols

    src_row = plsc.load_gather(nz_v, [row])
    part0 = plsc.load_gather(data_v, [src_row, col], mask=valid)

    dst_row = plsc.load_gather(idx0_v, [row])
    dst_ok = valid & (dst_row >= 0) & (dst_row < n_rows)
    dst_row = jnp.clip(dst_row, 0, n_rows - 1)
    plsc.store_scatter(out_v, [jnp.minimum(lanes, n_rows * n_cols - 1)],
                       zeros, mask=lanes < n_rows * n_cols)
    plsc.store_scatter(out_v, [dst_row * n_cols + col], part0, mask=dst_ok)

    pltpu.sync_copy(out_v, out_hbm)


def kernel(data, partitions, index0, index1):
    n_rows, n_cols = data.shape
    m0 = index0.shape[0]
    assert n_rows * n_cols <= _L and m0 * n_cols <= _L
    assert index1.shape[0] == 0

    body = functools.partial(_stitch_body, n_rows, n_cols, m0)
    out = pl.kernel(
        body,
        out_type=jax.ShapeDtypeStruct((n_rows * n_cols,), jnp.float32),
        mesh=plsc.VectorSubcoreMesh(
            core_axis_name="c", subcore_axis_name="s",
            num_cores=1, num_subcores=1,
        ),
        scratch_types=[
            pltpu.VMEM((n_rows,), jnp.int32),
            pltpu.VMEM((m0,), jnp.int32),
            pltpu.VMEM((n_rows, n_cols), jnp.float32),
            pltpu.VMEM((m0,), jnp.int32),
            pltpu.VMEM((n_rows * n_cols,), jnp.float32),
            pltpu.SemaphoreType.DMA,
        ],
        compiler_params=pltpu.CompilerParams(needs_layout_passes=False),
    )(partitions, index0, data)
    return out.reshape(n_rows, n_cols)


# speedup vs baseline: 1.1328x; 1.0724x over previous
import functools

import jax
import jax.numpy as jnp
from jax import lax
from jax.experimental import pallas as pl
from jax.experimental.pallas import tpu as pltpu
from jax.experimental.pallas import tpu_sc as plsc


def _scs_body(n_rows, n_cols, m0, part_hbm, idx0_hbm, data_hbm, out_hbm,
              part_s, idx0_s, data_s, nz_s, out_s, sem):
    copies = [pltpu.async_copy(part_hbm, part_s, sem),
              pltpu.async_copy(idx0_hbm, idx0_s, sem),
              pltpu.async_copy(data_hbm, data_s, sem)]
    for c in copies:
        c.wait()

    for i in range(m0):
        nz_s[i] = 0

    def step(i, cnt):
        hit = part_s[i] == 0

        @pl.when(hit & (cnt < m0))
        def _():
            nz_s[cnt] = i

        return cnt + jnp.where(hit, 1, 0)

    lax.fori_loop(0, n_rows, step, jnp.int32(0))

    for k in range(n_rows * n_cols):
        out_s[k] = 0.0

    def stitch(i, carry):
        r = nz_s[i]
        d = idx0_s[i]

        @pl.when((d >= 0) & (d < n_rows))
        def _():
            for j in range(n_cols):
                out_s[d * n_cols + j] = data_s[r, j]

        return carry

    lax.fori_loop(0, m0, stitch, jnp.int32(0))

    pltpu.sync_copy(out_s, out_hbm)


def kernel(data, partitions, index0, index1):
    n_rows, n_cols = data.shape
    m0 = index0.shape[0]
    assert index1.shape[0] == 0

    body = functools.partial(_scs_body, n_rows, n_cols, m0)
    out = pl.kernel(
        body,
        out_type=jax.ShapeDtypeStruct((n_rows * n_cols,), jnp.float32),
        mesh=plsc.ScalarSubcoreMesh(axis_name="c", num_cores=1),
        scratch_types=[
            pltpu.SMEM((n_rows,), jnp.int32),
            pltpu.SMEM((m0,), jnp.int32),
            pltpu.SMEM((n_rows, n_cols), jnp.float32),
            pltpu.SMEM((m0,), jnp.int32),
            pltpu.SMEM((n_rows * n_cols,), jnp.float32),
            pltpu.SemaphoreType.DMA,
        ],
        compiler_params=pltpu.CompilerParams(needs_layout_passes=False),
    )(partitions, index0, data)
    return out.reshape(n_rows, n_cols)
